# baseline (speedup 1.0000x reference)
import dataclasses
import functools

import jax
import jax.numpy as jnp
from jax import lax
from jax.experimental import pallas as pl
from jax.experimental.pallas import tpu as pltpu
from jax.experimental.pallas import tpu_sc as plsc

F32 = jnp.float32


def _sc_compiler_params():
    cp = pltpu.CompilerParams()
    if "needs_layout_passes" in pltpu.CompilerParams.__dataclass_fields__:
        cp = dataclasses.replace(cp, needs_layout_passes=False)
    return cp


def _gelu(x):
    return x * 0.5 * (1.0 + lax.erf(x * (2.0 ** -0.5)))


def _ln_rows(r, g, b, eps=1e-5):
    m = jnp.mean(r, axis=1, keepdims=True)
    c = r - m
    v = jnp.mean(c * c, axis=1, keepdims=True)
    return c * jax.lax.rsqrt(v + eps) * g + b


def _proj_body(nf, w_all, b_all, ts, td, pn):
    x = nf[...]
    h1 = jnp.dot(x, w_all[...], preferred_element_type=F32) + b_all[...]
    ts[...] = h1[:, 0:512].astype(jnp.bfloat16)
    td[...] = h1[:, 512:1024].astype(jnp.bfloat16)
    pn[...] = h1[:, 1024:1280]


def _node_proj(nf, w_all, b_all, blk):
    n, h = nf.shape
    grid = n // blk
    return pl.pallas_call(
        _proj_body,
        grid=(grid,),
        in_specs=[pl.BlockSpec((blk, h), lambda i: (i, 0)),
                  pl.BlockSpec(w_all.shape, lambda i: (0, 0)),
                  pl.BlockSpec(b_all.shape, lambda i: (0, 0))],
        out_specs=[pl.BlockSpec((blk, 2 * h), lambda i: (i, 0)),
                   pl.BlockSpec((blk, 2 * h), lambda i: (i, 0)),
                   pl.BlockSpec((blk, h), lambda i: (i, 0))],
        out_shape=[jax.ShapeDtypeStruct((n, 2 * h), jnp.bfloat16),
                   jax.ShapeDtypeStruct((n, 2 * h), jnp.bfloat16),
                   jax.ShapeDtypeStruct((n, h), F32)],
    )(nf, w_all, b_all)


def _edge_gather(tsrc, tdst, src1d, dst1d):
    e = src1d.shape[0]
    d = tsrc.shape[1]
    dt = tsrc.dtype
    nw = 32
    eps = e // nw
    gw = 40
    mesh = plsc.VectorSubcoreMesh(core_axis_name="core", subcore_axis_name="subcore")

    @functools.partial(
        pl.kernel,
        out_type=[jax.ShapeDtypeStruct((e, d), dt),
                  jax.ShapeDtypeStruct((e, d), dt)],
        mesh=mesh,
        scratch_types=[pltpu.VMEM((2, gw), jnp.int32),
                       pltpu.VMEM((2, gw), jnp.int32),
                       pltpu.VMEM((2, gw, d), dt),
                       pltpu.VMEM((2, gw, d), dt),
                       pltpu.SemaphoreType.DMA,
                       pltpu.SemaphoreType.DMA,
                       pltpu.SemaphoreType.DMA,
                       pltpu.SemaphoreType.DMA],
        compiler_params=_sc_compiler_params(),
    )
    def gather_kernel(tsrc_hbm, tdst_hbm, src_hbm, dst_hbm, osrc_hbm, odst_hbm,
                      idx1, idx2, rows1, rows2, sem1a, sem1b_, sem2a, sem2b_):
        t = lax.axis_index("subcore") * 2 + lax.axis_index("core")
        base0 = t * eps
        nsteps = eps // gw

        idx1b = (idx1.at[0], idx1.at[1])
        idx2b = (idx2.at[0], idx2.at[1])
        rows1b = (rows1.at[0], rows1.at[1])
        rows2b = (rows2.at[0], rows2.at[1])
        sem1b = (sem1a, sem1b_)
        sem2b = (sem2a, sem2b_)

        def start(s, db):
            b = base0 + s * gw
            pltpu.sync_copy(src_hbm.at[pl.ds(b, gw)], idx1b[db])
            pltpu.sync_copy(dst_hbm.at[pl.ds(b, gw)], idx2b[db])
            pltpu.async_copy(tsrc_hbm.at[idx1b[db]], rows1b[db], sem1b[db])
            pltpu.async_copy(tdst_hbm.at[idx2b[db]], rows2b[db], sem2b[db])

        def finish(s, db):
            b = base0 + s * gw
            pltpu.make_async_copy(tsrc_hbm.at[idx1b[db]], rows1b[db],
                                  sem1b[db]).wait()
            pltpu.make_async_copy(tdst_hbm.at[idx2b[db]], rows2b[db],
                                  sem2b[db]).wait()
            pltpu.sync_copy(rows1b[db], osrc_hbm.at[pl.ds(b, gw)])
            pltpu.sync_copy(rows2b[db], odst_hbm.at[pl.ds(b, gw)])

        start(0, 0)
        rem = nsteps % 2

        @pl.loop(0, nsteps - rem, step=2)
        def _(m):
            for db in range(2):
                mm = m + db

                @pl.when(mm + 1 < nsteps)
                def _():
                    start(mm + 1, 1 - db)

                finish(mm, db)

        if rem:
            finish(nsteps - 1, (nsteps - 1) % 2)

    return gather_kernel(tsrc, tdst, src1d, dst1d)


def _edge_body(gs, gd, ef, wqm, wqe, mw2, ew2, eb2, lg, lb, msgt, ne):
    x = ef[...]
    gsv = gs[...].astype(F32)
    gdv = gd[...].astype(F32)
    pre_m = (gsv[:, 0:256] + gdv[:, 0:256]
             + jnp.dot(x, wqm[...], preferred_element_type=F32))
    h_m = _gelu(pre_m)
    msgt[...] = lax.dot_general(mw2[...], h_m,
                                dimension_numbers=(((0,), (1,)), ((), ())),
                                preferred_element_type=F32)
    pre_e = (gsv[:, 256:512] + gdv[:, 256:512]
             + jnp.dot(x, wqe[...], preferred_element_type=F32))
    h_e = _gelu(pre_e)
    eu = jnp.dot(h_e, ew2[...], preferred_element_type=F32) + eb2[...]
    ne[...] = _ln_rows(x + eu, lg[...], lb[...])


def _edge_mlp(gs, gd, ef, wqm, wqe, mw2, ew2, eb2, lg, lb, blk):
    e, h = ef.shape
    grid = e // blk
    full = lambda a: pl.BlockSpec(a.shape, lambda i: (0, 0))
    rowblk = pl.BlockSpec((blk, h), lambda i: (i, 0))
    wideblk = pl.BlockSpec((blk, 2 * h), lambda i: (i, 0))
    return pl.pallas_call(
        _edge_body,
        grid=(grid,),
        in_specs=[wideblk, wideblk, rowblk,
                  full(wqm), full(wqe), full(mw2), full(ew2),
                  full(eb2), full(lg), full(lb)],
        out_specs=[pl.BlockSpec((h, blk), lambda i: (0, i)), rowblk],
        out_shape=[jax.ShapeDtypeStruct((h, e), F32),
                   jax.ShapeDtypeStruct((e, h), F32)],
    )(gs, gd, ef, wqm, wqe, mw2, ew2, eb2, lg, lb)


def _scatter_agg(msgt, dst1d, n_nodes):
    h, e = msgt.shape
    ngroups = 32
    gw = h // ngroups
    npad = ((n_nodes + 7) // 8) * 8 + 48
    acc_len = npad * gw
    dlen = ((n_nodes + 15) // 16) * 16
    ch = 640
    nch = e // ch
    ndw = 16
    eps = e // ndw
    mesh = plsc.VectorSubcoreMesh(core_axis_name="core", subcore_axis_name="subcore")

    @functools.partial(
        pl.kernel,
        out_type=[jax.ShapeDtypeStruct((32 * acc_len,), F32),
                  jax.ShapeDtypeStruct((32 * dlen,), F32)],
        mesh=mesh,
        scratch_types=[pltpu.VMEM((acc_len,), F32),
                       pltpu.VMEM((dlen,), F32),
                       pltpu.VMEM((gw, ch), F32),
                       pltpu.VMEM((gw, ch), F32),
                       pltpu.VMEM((gw, ch), F32),
                       pltpu.VMEM((gw, ch), F32),
                       pltpu.VMEM((ch,), jnp.int32),
                       pltpu.VMEM((ch,), jnp.int32),
                       pltpu.VMEM((ch,), jnp.int32),
                       pltpu.VMEM((ch,), jnp.int32),
                       pltpu.VMEM((eps,), jnp.int32),
                       pltpu.SemaphoreType.DMA,
                       pltpu.SemaphoreType.DMA,
                       pltpu.SemaphoreType.DMA,
                       pltpu.SemaphoreType.DMA,
                       pltpu.SemaphoreType.DMA,
                       pltpu.SemaphoreType.DMA,
                       pltpu.SemaphoreType.DMA,
                       pltpu.SemaphoreType.DMA],
        compiler_params=_sc_compiler_params(),
    )
    def scatter_kernel(msgt_hbm, dst_hbm, agg_hbm, deg_hbm,
                       acc1, dhist, vals0, vals1, vals2, vals3,
                       raw0, raw1, raw2, raw3, rawd,
                       sg0, sg1, sg2, sg3, sr0, sr1, sr2, sr3):
        t = lax.axis_index("subcore") * 2 + lax.axis_index("core")
        iota = lax.iota(jnp.int32, 16)
        zero16 = jnp.zeros((16,), F32)
        ones16 = jnp.ones((16,), F32)

        valsb = (vals0, vals1, vals2, vals3)
        rawb = (raw0, raw1, raw2, raw3)
        semgb = (sg0, sg1, sg2, sg3)
        semrb = (sr0, sr1, sr2, sr3)

        @pl.loop(0, acc_len, step=16)
        def _(i):
            acc1[pl.ds(i, 16)] = zero16

        @pl.loop(0, dlen, step=16)
        def _(i):
            dhist[pl.ds(i, 16)] = zero16

        def issue(mm, b):
            e0 = mm * ch
            pltpu.async_copy(msgt_hbm.at[pl.ds(t * gw, gw), pl.ds(e0, ch)],
                             valsb[b], semgb[b])
            pltpu.async_copy(dst_hbm.at[pl.ds(e0, ch)], rawb[b], semrb[b])

        def wait(mm, b):
            e0 = mm * ch
            pltpu.make_async_copy(msgt_hbm.at[pl.ds(t * gw, gw), pl.ds(e0, ch)],
                                  valsb[b], semgb[b]).wait()
            pltpu.make_async_copy(dst_hbm.at[pl.ds(e0, ch)], rawb[b], semrb[b]).wait()

        def compute(b):
            @pl.loop(0, ch, step=16)
            def _(j):
                offs0 = rawb[b][pl.ds(j, 16)] * gw
                for r in range(gw):
                    vv = valsb[b][r, pl.ds(j, 16)]
                    plsc.addupdate_scatter(acc1, [offs0 + r], vv)

        issue(0, 0)
        issue(1, 1)
        issue(2, 2)

        main = (nch // 4) * 4 - 4

        @pl.loop(0, main + 4, step=4)
        def _(m):
            for db in range(4):
                mm = m + db

                @pl.when(mm + 3 < nch)
                def _():
                    issue(mm + 3, (db + 3) % 4)

                wait(mm, db)
                compute(db)

        for mm in range(main + 4, nch):
            wait(mm, mm % 4)
            compute(mm % 4)

        full16 = (eps // 16) * 16

        @pl.when(t < ndw)
        def _():
            e0d = t * eps
            pltpu.sync_copy(dst_hbm.at[pl.ds(e0d, eps)], rawd)

            @pl.loop(0, full16, step=16)
            def _(j):
                plsc.addupdate_scatter(dhist, [rawd[pl.ds(j, 16)]], ones16)

        if eps != full16:
            @pl.when(t < ndw)
            def _():
                tail = rawd[pl.ds(eps - 16, 16)]
                plsc.addupdate_scatter(dhist, [tail], ones16,
                                       mask=iota >= (16 - (eps - full16)))

        pltpu.sync_copy(acc1, agg_hbm.at[pl.ds(t * acc_len, acc_len)])
        pltpu.sync_copy(dhist, deg_hbm.at[pl.ds(t * dlen, dlen)])

    agg_f, deg_f = scatter_kernel(msgt, dst1d)
    agg = agg_f.reshape(32, npad, gw).transpose(1, 0, 2).reshape(npad, h)[:n_nodes]
    deg_t = deg_f.reshape(32, dlen)[:, :n_nodes].T
    return agg, deg_t


def _node_body(nf, pn, agg0, agg1, deg0, deg1, w1d, nb1, mb2r, nw2, nb2,
               lg, lb, out):
    degree = (jnp.sum(deg0[...], axis=1, keepdims=True)
              + jnp.sum(deg1[...], axis=1, keepdims=True))
    aggs = agg0[...] + agg1[...]
    aggn = (aggs + degree * mb2r[...]) / (degree + 1e-8)
    pre = pn[...] + jnp.dot(aggn, w1d[...], preferred_element_type=F32) + nb1[...]
    hh = _gelu(pre)
    upd = jnp.dot(hh, nw2[...], preferred_element_type=F32) + nb2[...]
    out[...] = _ln_rows(nf[...] + upd, lg[...], lb[...])


def _node_update(nf, pn, agg0, agg1, deg0, deg1, w1d, nb1, mb2r, nw2, nb2,
                 lg, lb, blk):
    n, h = nf.shape
    grid = n // blk
    full = lambda a: pl.BlockSpec(a.shape, lambda i: (0, 0))
    rowblk = pl.BlockSpec((blk, h), lambda i: (i, 0))
    degblk = pl.BlockSpec((blk, 32), lambda i: (i, 0))
    return pl.pallas_call(
        _node_body,
        grid=(grid,),
        in_specs=[rowblk, rowblk, rowblk, rowblk, degblk, degblk,
                  full(w1d), full(nb1), full(mb2r), full(nw2), full(nb2),
                  full(lg), full(lb)],
        out_specs=rowblk,
        out_shape=jax.ShapeDtypeStruct((n, h), F32),
    )(nf, pn, agg0, agg1, deg0, deg1, w1d, nb1, mb2r, nw2, nb2, lg, lb)


def kernel(node_features, edge_features, edge_index, params):
    p = params
    n, h = node_features.shape
    e = edge_features.shape[0]

    w_all = jnp.concatenate(
        [p['mW1'][0:h], p['eW1'][0:h],
         p['mW1'][h:2 * h], p['eW1'][h:2 * h],
         p['nW1'][0:h]], axis=1)
    b_all = jnp.concatenate(
        [p['mb1'], p['eb1'], jnp.zeros((3 * h,), F32)]).reshape(1, 5 * h)

    tsrc, tdst, pn = _node_proj(node_features, w_all, b_all, blk=2000)

    ts32 = lax.bitcast_convert_type(tsrc.reshape(n, h, 2), jnp.int32)
    td32 = lax.bitcast_convert_type(tdst.reshape(n, h, 2), jnp.int32)

    src = edge_index[0]
    dst = edge_index[1]

    h0 = 79360
    cut_args = lambda lo, hi: (src[lo:hi], dst[lo:hi], edge_features[lo:hi])

    halves = []
    for lo, hi in ((0, h0), (h0, e)):
        s_h, d_h, ef_h = cut_args(lo, hi)
        gs32, gd32 = _edge_gather(ts32, td32, s_h, d_h)
        gs = lax.bitcast_convert_type(gs32, jnp.bfloat16).reshape(hi - lo, 2 * h)
        gd = lax.bitcast_convert_type(gd32, jnp.bfloat16).reshape(hi - lo, 2 * h)
        msgt, ne = _edge_mlp(
            gs, gd, ef_h,
            p['mW1'][2 * h:3 * h], p['eW1'][2 * h:3 * h],
            p['mW2'], p['eW2'], p['eb2'].reshape(1, h),
            p['edge_ln_g'].reshape(1, h), p['edge_ln_b'].reshape(1, h),
            blk=1280)
        agg_h, deg_h = _scatter_agg(msgt, d_h, n)
        halves.append((ne, agg_h, deg_h))

    new_edge = jnp.concatenate([halves[0][0], halves[1][0]], axis=0)

    new_node = _node_update(
        node_features, pn, halves[0][1], halves[1][1],
        halves[0][2], halves[1][2],
        p['nW1'][h:2 * h], p['nb1'].reshape(1, h), p['mb2'].reshape(1, h),
        p['nW2'], p['nb2'].reshape(1, h),
        p['node_ln_g'].reshape(1, h), p['node_ln_b'].reshape(1, h), blk=1000)

    return (new_node, new_edge)


# speedup vs baseline: 2.6558x; 2.6558x over previous
import dataclasses
import functools

import jax
import jax.numpy as jnp
from jax import lax
from jax.experimental import pallas as pl
from jax.experimental.pallas import tpu as pltpu
from jax.experimental.pallas import tpu_sc as plsc

F32 = jnp.float32


def _sc_compiler_params():
    cp = pltpu.CompilerParams()
    if "needs_layout_passes" in pltpu.CompilerParams.__dataclass_fields__:
        cp = dataclasses.replace(cp, needs_layout_passes=False)
    return cp


def _gelu(x):
    return x * 0.5 * (1.0 + lax.erf(x * (2.0 ** -0.5)))


def _ln_rows(r, g, b, eps=1e-5):
    m = jnp.mean(r, axis=1, keepdims=True)
    c = r - m
    v = jnp.mean(c * c, axis=1, keepdims=True)
    return c * jax.lax.rsqrt(v + eps) * g + b


def _proj_body(nf, w, b, tsrc, tdst, pn):
    pr = jnp.dot(nf[...], w[...], preferred_element_type=F32) + b[...]
    tsrc[...] = pr[:, 0:512]
    tdst[...] = pr[:, 512:1024]
    pn[...] = pr[:, 1024:1280]


def _node_proj(nf, w_all, b_all, blk):
    n, h = nf.shape
    grid = n // blk
    full = lambda a: pl.BlockSpec(a.shape, lambda i: (0, 0))
    return pl.pallas_call(
        _proj_body,
        grid=(grid,),
        in_specs=[pl.BlockSpec((blk, h), lambda i: (i, 0)),
                  full(w_all), full(b_all)],
        out_specs=[pl.BlockSpec((blk, 2 * h), lambda i: (i, 0)),
                   pl.BlockSpec((blk, 2 * h), lambda i: (i, 0)),
                   pl.BlockSpec((blk, h), lambda i: (i, 0))],
        out_shape=[jax.ShapeDtypeStruct((n, 2 * h), F32),
                   jax.ShapeDtypeStruct((n, 2 * h), F32),
                   jax.ShapeDtypeStruct((n, h), F32)],
    )(nf, w_all, b_all)


def _edge_gather(tsrc, tdst, src1d, dst1d):
    e = src1d.shape[0]
    d = tsrc.shape[1]
    dt = tsrc.dtype
    nw = 32
    eps = e // nw
    gw = 40
    mesh = plsc.VectorSubcoreMesh(core_axis_name="core", subcore_axis_name="subcore")

    @functools.partial(
        pl.kernel,
        out_type=[jax.ShapeDtypeStruct((e, d), dt),
                  jax.ShapeDtypeStruct((e, d), dt)],
        mesh=mesh,
        scratch_types=[pltpu.VMEM((2, gw), jnp.int32),
                       pltpu.VMEM((2, gw), jnp.int32),
                       pltpu.VMEM((2, gw, d), dt),
                       pltpu.VMEM((2, gw, d), dt),
                       pltpu.SemaphoreType.DMA,
                       pltpu.SemaphoreType.DMA,
                       pltpu.SemaphoreType.DMA,
                       pltpu.SemaphoreType.DMA],
        compiler_params=_sc_compiler_params(),
    )
    def gather_kernel(tsrc_hbm, tdst_hbm, src_hbm, dst_hbm, osrc_hbm, odst_hbm,
                      idx1, idx2, rows1, rows2, sem1a, sem1b_, sem2a, sem2b_):
        t = lax.axis_index("subcore") * 2 + lax.axis_index("core")
        base0 = t * eps
        nsteps = eps // gw

        idx1b = (idx1.at[0], idx1.at[1])
        idx2b = (idx2.at[0], idx2.at[1])
        rows1b = (rows1.at[0], rows1.at[1])
        rows2b = (rows2.at[0], rows2.at[1])
        sem1b = (sem1a, sem1b_)
        sem2b = (sem2a, sem2b_)

        def start(s, db):
            b = base0 + s * gw
            pltpu.sync_copy(src_hbm.at[pl.ds(b, gw)], idx1b[db])
            pltpu.sync_copy(dst_hbm.at[pl.ds(b, gw)], idx2b[db])
            pltpu.async_copy(tsrc_hbm.at[idx1b[db]], rows1b[db], sem1b[db])
            pltpu.async_copy(tdst_hbm.at[idx2b[db]], rows2b[db], sem2b[db])

        def finish(s, db):
            b = base0 + s * gw
            pltpu.make_async_copy(tsrc_hbm.at[idx1b[db]], rows1b[db],
                                  sem1b[db]).wait()
            pltpu.make_async_copy(tdst_hbm.at[idx2b[db]], rows2b[db],
                                  sem2b[db]).wait()
            pltpu.sync_copy(rows1b[db], osrc_hbm.at[pl.ds(b, gw)])
            pltpu.sync_copy(rows2b[db], odst_hbm.at[pl.ds(b, gw)])

        start(0, 0)
        rem = nsteps % 2

        @pl.loop(0, nsteps - rem, step=2)
        def _(m):
            for db in range(2):
                mm = m + db

                @pl.when(mm + 1 < nsteps)
                def _():
                    start(mm + 1, 1 - db)

                finish(mm, db)

        if rem:
            finish(nsteps - 1, (nsteps - 1) % 2)

    return gather_kernel(tsrc, tdst, src1d, dst1d)


def _edge_body(gs, gd, ef, wmq, weq, mw2, ew2, eb2, lg, lb, msgt, ne):
    x = ef[...]
    gsv = gs[...]
    gdv = gd[...]
    pre_m = (gsv[:, 0:256] + gdv[:, 0:256]
             + jnp.dot(x, wmq[...], preferred_element_type=F32))
    h_m = _gelu(pre_m)
    msgt[...] = lax.dot_general(mw2[...], h_m,
                                dimension_numbers=(((0,), (1,)), ((), ())),
                                preferred_element_type=F32)
    pre_e = (gsv[:, 256:512] + gdv[:, 256:512]
             + jnp.dot(x, weq[...], preferred_element_type=F32))
    h_e = _gelu(pre_e)
    eu = jnp.dot(h_e, ew2[...], preferred_element_type=F32) + eb2[...]
    ne[...] = _ln_rows(x + eu, lg[...], lb[...])


def _edge_mlp(gs, gd, ef, wmq, weq, mw2, ew2, eb2, lg, lb, blk):
    e, h = ef.shape
    grid = e // blk
    full = lambda a: pl.BlockSpec(a.shape, lambda i: (0, 0))
    rowblk = pl.BlockSpec((blk, h), lambda i: (i, 0))
    wideblk = pl.BlockSpec((blk, 2 * h), lambda i: (i, 0))
    return pl.pallas_call(
        _edge_body,
        grid=(grid,),
        in_specs=[wideblk, wideblk, rowblk,
                  full(wmq), full(weq),
                  full(mw2), full(ew2), full(eb2), full(lg), full(lb)],
        out_specs=[pl.BlockSpec((h, blk), lambda i: (0, i)), rowblk],
        out_shape=[jax.ShapeDtypeStruct((h, e), F32),
                   jax.ShapeDtypeStruct((e, h), F32)],
    )(gs, gd, ef, wmq, weq, mw2, ew2, eb2, lg, lb)


def _scatter_agg(msgt, dst1d, n_nodes):
    h, e = msgt.shape
    ngroups = 32
    gw = h // ngroups
    npad = ((n_nodes + 7) // 8) * 8 + 48
    acc_len = npad * gw
    dlen = ((n_nodes + 15) // 16) * 16
    ch = 640
    nch = e // ch
    ndw = 16
    eps = e // ndw
    mesh = plsc.VectorSubcoreMesh(core_axis_name="core", subcore_axis_name="subcore")

    @functools.partial(
        pl.kernel,
        out_type=[jax.ShapeDtypeStruct((32 * acc_len,), F32),
                  jax.ShapeDtypeStruct((32 * dlen,), F32)],
        mesh=mesh,
        scratch_types=[pltpu.VMEM((acc_len,), F32),
                       pltpu.VMEM((dlen,), F32),
                       pltpu.VMEM((gw, ch), F32),
                       pltpu.VMEM((gw, ch), F32),
                       pltpu.VMEM((gw, ch), F32),
                       pltpu.VMEM((gw, ch), F32),
                       pltpu.VMEM((ch,), jnp.int32),
                       pltpu.VMEM((ch,), jnp.int32),
                       pltpu.VMEM((ch,), jnp.int32),
                       pltpu.VMEM((ch,), jnp.int32),
                       pltpu.VMEM((eps,), jnp.int32),
                       pltpu.SemaphoreType.DMA,
                       pltpu.SemaphoreType.DMA,
                       pltpu.SemaphoreType.DMA,
                       pltpu.SemaphoreType.DMA,
                       pltpu.SemaphoreType.DMA,
                       pltpu.SemaphoreType.DMA,
                       pltpu.SemaphoreType.DMA,
                       pltpu.SemaphoreType.DMA],
        compiler_params=_sc_compiler_params(),
    )
    def scatter_kernel(msgt_hbm, dst_hbm, agg_hbm, deg_hbm,
                       acc1, dhist, vals0, vals1, vals2, vals3,
                       raw0, raw1, raw2, raw3, rawd,
                       sg0, sg1, sg2, sg3, sr0, sr1, sr2, sr3):
        t = lax.axis_index("subcore") * 2 + lax.axis_index("core")
        iota = lax.iota(jnp.int32, 16)
        zero16 = jnp.zeros((16,), F32)
        ones16 = jnp.ones((16,), F32)

        valsb = (vals0, vals1, vals2, vals3)
        rawb = (raw0, raw1, raw2, raw3)
        semgb = (sg0, sg1, sg2, sg3)
        semrb = (sr0, sr1, sr2, sr3)

        @pl.loop(0, acc_len, step=16)
        def _(i):
            acc1[pl.ds(i, 16)] = zero16

        @pl.loop(0, dlen, step=16)
        def _(i):
            dhist[pl.ds(i, 16)] = zero16

        def issue(mm, b):
            e0 = mm * ch
            pltpu.async_copy(msgt_hbm.at[pl.ds(t * gw, gw), pl.ds(e0, ch)],
                             valsb[b], semgb[b])
            pltpu.async_copy(dst_hbm.at[pl.ds(e0, ch)], rawb[b], semrb[b])

        def wait(mm, b):
            e0 = mm * ch
            pltpu.make_async_copy(msgt_hbm.at[pl.ds(t * gw, gw), pl.ds(e0, ch)],
                                  valsb[b], semgb[b]).wait()
            pltpu.make_async_copy(dst_hbm.at[pl.ds(e0, ch)], rawb[b], semrb[b]).wait()

        def compute(b):
            @pl.loop(0, ch, step=16)
            def _(j):
                offs0 = rawb[b][pl.ds(j, 16)] * gw
                for r in range(gw):
                    vv = valsb[b][r, pl.ds(j, 16)]
                    plsc.addupdate_scatter(acc1, [offs0 + r], vv)

        issue(0, 0)
        issue(1, 1)
        issue(2, 2)

        main = (nch // 4) * 4 - 4

        @pl.loop(0, main + 4, step=4)
        def _(m):
            for db in range(4):
                mm = m + db

                @pl.when(mm + 3 < nch)
                def _():
                    issue(mm + 3, (db + 3) % 4)

                wait(mm, db)
                compute(db)

        for mm in range(main + 4, nch):
            wait(mm, mm % 4)
            compute(mm % 4)

        full16 = (eps // 16) * 16

        @pl.when(t < ndw)
        def _():
            e0d = t * eps
            pltpu.sync_copy(dst_hbm.at[pl.ds(e0d, eps)], rawd)

            @pl.loop(0, full16, step=16)
            def _(j):
                plsc.addupdate_scatter(dhist, [rawd[pl.ds(j, 16)]], ones16)

        if eps != full16:
            @pl.when(t < ndw)
            def _():
                tail = rawd[pl.ds(eps - 16, 16)]
                plsc.addupdate_scatter(dhist, [tail], ones16,
                                       mask=iota >= (16 - (eps - full16)))

        pltpu.sync_copy(acc1, agg_hbm.at[pl.ds(t * acc_len, acc_len)])
        pltpu.sync_copy(dhist, deg_hbm.at[pl.ds(t * dlen, dlen)])

    agg_f, deg_f = scatter_kernel(msgt, dst1d)
    agg = agg_f.reshape(32, npad, gw).transpose(1, 0, 2).reshape(npad, h)[:n_nodes]
    deg_t = deg_f.reshape(32, dlen)[:, :n_nodes].T
    return agg, deg_t


def _node_body(pn, nf, agg, deg, w1a, nb1, mb2r, nw2, nb2, lg, lb, out):
    xn = nf[...]
    degree = jnp.sum(deg[...], axis=1, keepdims=True)
    aggn = (agg[...] + degree * mb2r[...]) / (degree + 1e-8)
    pre = (pn[...] + jnp.dot(aggn, w1a[...], preferred_element_type=F32)
           + nb1[...])
    hh = _gelu(pre)
    upd = jnp.dot(hh, nw2[...], preferred_element_type=F32) + nb2[...]
    out[...] = _ln_rows(xn + upd, lg[...], lb[...])


def _node_update(pn, nf, agg, deg, w1a, nb1, mb2r, nw2, nb2, lg, lb, blk):
    n, h = nf.shape
    grid = n // blk
    full = lambda a: pl.BlockSpec(a.shape, lambda i: (0, 0))
    rowblk = pl.BlockSpec((blk, h), lambda i: (i, 0))
    degblk = pl.BlockSpec((blk, 32), lambda i: (i, 0))
    return pl.pallas_call(
        _node_body,
        grid=(grid,),
        in_specs=[rowblk, rowblk, rowblk, degblk,
                  full(w1a), full(nb1), full(mb2r), full(nw2), full(nb2),
                  full(lg), full(lb)],
        out_specs=rowblk,
        out_shape=jax.ShapeDtypeStruct((n, h), F32),
    )(pn, nf, agg, deg, w1a, nb1, mb2r, nw2, nb2, lg, lb)


def kernel(node_features, edge_features, edge_index, params):
    p = params
    n, h = node_features.shape
    e = edge_features.shape[0]

    w_all = jnp.concatenate(
        [p['mW1'][0:h], p['eW1'][0:h],
         p['mW1'][h:2 * h], p['eW1'][h:2 * h],
         p['nW1'][0:h]], axis=1)
    b_all = jnp.concatenate(
        [p['mb1'], p['eb1'], jnp.zeros((3 * h,), F32)]).reshape(1, 5 * h)

    tsrc, tdst, pn = _node_proj(node_features, w_all, b_all, blk=2000)

    src = edge_index[0]
    dst = edge_index[1]

    gs, gd = _edge_gather(tsrc, tdst, src, dst)

    msgt, new_edge = _edge_mlp(
        gs, gd, edge_features,
        p['mW1'][2 * h:3 * h], p['eW1'][2 * h:3 * h],
        p['mW2'], p['eW2'], p['eb2'].reshape(1, h),
        p['edge_ln_g'].reshape(1, h), p['edge_ln_b'].reshape(1, h),
        blk=1280)

    agg, deg_t = _scatter_agg(msgt, dst, n)

    new_node = _node_update(
        pn, node_features, agg, deg_t,
        p['nW1'][h:2 * h], p['nb1'].reshape(1, h), p['mb2'].reshape(1, h),
        p['nW2'], p['nb2'].reshape(1, h),
        p['node_ln_g'].reshape(1, h), p['node_ln_b'].reshape(1, h), blk=1000)

    return (new_node, new_edge)
